# baseline (speedup 1.0000x reference)
import functools

import jax
import jax.numpy as jnp
from jax import lax
from jax.experimental import pallas as pl
from jax.experimental.pallas import tpu as pltpu
from jax.experimental.pallas import tpu_sc as plsc

D = 64
NC, NS, L = 2, 16, 16
NW = NC * NS
B = 4096
H = 200
G = 128
NBT = B // G
NGRP = H * NBT
PER_W = NGRP // NW

_mesh = plsc.VectorSubcoreMesh(core_axis_name="c", subcore_axis_name="s")


@functools.partial(
    pl.kernel,
    out_type=jax.ShapeDtypeStruct((H, D // 8, NBT, 8, G), jnp.float32),
    mesh=_mesh,
    scratch_types=[
        pltpu.VMEM((G,), jnp.int32),
        pltpu.VMEM((G,), jnp.int32),
        pltpu.VMEM((G,), jnp.int32),
        pltpu.VMEM((G,), jnp.int32),
        pltpu.VMEM((G, 128), jnp.float32),
        pltpu.VMEM((D, G), jnp.float32),
        pltpu.SemaphoreType.DMA,
        pltpu.SemaphoreType.DMA,
        pltpu.SemaphoreType.DMA,
    ],
    compiler_params=pltpu.CompilerParams(
        use_tc_tiling_on_sc=False, needs_layout_passes=False),
)
def _emb_lookup(xf_hbm, wv_hbm, out_hbm, idx_v, idxf_v, colb_v, rowi_v,
                rows_v, outt_v, s_idx, s_g, s_o):
    wid = lax.axis_index("s") * NC + lax.axis_index("c")
    g0 = wid * PER_W

    for k in range(G // L):
        rowi_v[pl.ds(k * L, L)] = lax.iota(jnp.int32, L) + (k * L)

    @pl.loop(0, PER_W)
    def _grp(j):
        g = g0 + j
        h = g >> 5
        bt = g & (NBT - 1)
        q0 = h * B + bt * G

        pltpu.async_copy(xf_hbm.at[pl.ds(q0, G)], idx_v, s_idx).wait()
        for k in range(G // L):
            v = idx_v[pl.ds(k * L, L)]
            idxf_v[pl.ds(k * L, L)] = lax.shift_right_logical(v, 1)
            colb_v[pl.ds(k * L, L)] = lax.shift_left(
                lax.bitwise_and(v, 1), 6)
        pltpu.async_copy(wv_hbm.at[idxf_v], rows_v, s_g).wait()

        @pl.loop(0, D)
        def _d(d):
            for k in range(G // L):
                row = rowi_v[pl.ds(k * L, L)]
                col = colb_v[pl.ds(k * L, L)] + d
                outt_v[d, pl.ds(k * L, L)] = plsc.load_gather(
                    rows_v, [row, col])

        for dt in range(D // 8):
            pltpu.async_copy(
                outt_v.at[pl.ds(dt * 8, 8)], out_hbm.at[h, dt, bt], s_o)
        for dt in range(D // 8):
            pltpu.make_async_copy(
                outt_v.at[pl.ds(dt * 8, 8)], out_hbm.at[h, dt, bt], s_o
            ).wait()


def kernel(x, weight):
    xf = x.T.reshape(B * H)
    wv = weight.reshape(500000, 128)
    out5 = _emb_lookup(xf, wv)
    return out5.transpose((2, 4, 0, 1, 3)).reshape(B, H, D)


# speedup vs baseline: 1.5631x; 1.5631x over previous
import functools

import jax
import jax.numpy as jnp
from jax import lax
from jax.experimental import pallas as pl
from jax.experimental.pallas import tpu as pltpu
from jax.experimental.pallas import tpu_sc as plsc

D = 64
NC, NS, L = 2, 16, 16
NW = NC * NS
B = 4096
H = 200
G = 256
NBP = B // G
NGRP = H * NBP
PER_W = NGRP // NW
NV = G // L

_mesh = plsc.VectorSubcoreMesh(core_axis_name="c", subcore_axis_name="s")


@functools.partial(
    pl.kernel,
    out_type=jax.ShapeDtypeStruct((H, D // 8, B // 128, 8, 128), jnp.float32),
    mesh=_mesh,
    scratch_types=[
        pltpu.VMEM((2, G), jnp.int32),
        pltpu.VMEM((2, G), jnp.int32),
        pltpu.VMEM((2, G), jnp.int32),
        pltpu.VMEM((G,), jnp.int32),
        pltpu.VMEM((2, G, 128), jnp.float32),
        pltpu.VMEM((2, 8, 2, 8, 128), jnp.float32),
        pltpu.SemaphoreType.DMA((2,)),
        pltpu.SemaphoreType.DMA((2,)),
        pltpu.SemaphoreType.DMA((2,)),
    ],
    compiler_params=pltpu.CompilerParams(
        use_tc_tiling_on_sc=False, needs_layout_passes=False),
)
def _emb_lookup(xf_hbm, wv_hbm, out_hbm, idx_v, idxf_v, colb_v, rowi_v,
                rows_v, outt_v, s_idx, s_g, s_o):
    wid = lax.axis_index("s") * NC + lax.axis_index("c")
    j0 = wid * PER_W

    for k in range(NV):
        rowi_v[pl.ds(k * L, L)] = lax.iota(jnp.int32, L) + (k * L)

    def q_of(j):
        g2 = j0 + j
        h = lax.shift_right_logical(g2, 4)
        btp = lax.bitwise_and(g2, NBP - 1)
        return h, btp, h * B + btp * G

    def start_idx(p, j):
        _, _, q0 = q_of(j)
        pltpu.async_copy(xf_hbm.at[pl.ds(q0, G)], idx_v.at[p], s_idx.at[p])

    def wait_idx(p):
        pltpu.make_async_copy(
            xf_hbm.at[pl.ds(0, G)], idx_v.at[p], s_idx.at[p]).wait()

    def fuse(p):
        for k in range(NV):
            v = idx_v[p, pl.ds(k * L, L)]
            idxf_v[p, pl.ds(k * L, L)] = lax.shift_right_logical(v, 1)
            colb_v[p, pl.ds(k * L, L)] = lax.shift_left(
                lax.bitwise_and(v, 1), 6)

    def start_gather(p):
        pltpu.async_copy(wv_hbm.at[idxf_v.at[p]], rows_v.at[p], s_g.at[p])

    def wait_gather(p):
        pltpu.make_async_copy(
            wv_hbm.at[idxf_v.at[p]], rows_v.at[p], s_g.at[p]).wait()

    def transpose(p):
        @pl.loop(0, D // 8)
        def _dt(dt):
            for btl in range(2):
                for k2 in range(8):
                    off = btl * 128 + k2 * L
                    row = rowi_v[pl.ds(off, L)]
                    cb = colb_v[p, pl.ds(off, L)]
                    for dp in range(8):
                        col = cb + (dt * 8 + dp)
                        outt_v[p, dt, btl, dp, pl.ds(k2 * L, L)] = (
                            plsc.load_gather(rows_v.at[p], [row, col]))

    def start_write(p, j):
        h, btp, _ = q_of(j)
        pltpu.async_copy(
            outt_v.at[p], out_hbm.at[h, :, pl.ds(btp * 2, 2)], s_o.at[p])

    def wait_write(p, j):
        h, btp, _ = q_of(j)
        pltpu.make_async_copy(
            outt_v.at[p], out_hbm.at[h, :, pl.ds(btp * 2, 2)], s_o.at[p]
        ).wait()

    start_idx(0, 0)
    wait_idx(0)
    fuse(0)
    start_gather(0)
    start_idx(1, 1)

    for j in (0, 1):
        p, pn = j % 2, (j + 1) % 2
        wait_idx(pn)
        fuse(pn)
        start_gather(pn)
        start_idx(p, j + 2)
        wait_gather(p)
        transpose(p)
        start_write(p, j)

    @pl.loop(0, (PER_W - 4) // 2)
    def _grp(t):
        jj = 2 + t * 2
        for r in range(2):
            j = jj + r
            p, pn = r, 1 - r
            wait_idx(pn)
            fuse(pn)
            start_gather(pn)
            start_idx(p, j + 2)
            wait_gather(p)
            wait_write(p, j - 2)
            transpose(p)
            start_write(p, j)

    j = PER_W - 2
    wait_idx(1)
    fuse(1)
    start_gather(1)
    wait_gather(0)
    wait_write(0, j - 2)
    transpose(0)
    start_write(0, j)

    j = PER_W - 1
    wait_gather(1)
    wait_write(1, j - 2)
    transpose(1)
    start_write(1, j)

    wait_write(0, PER_W - 2)
    wait_write(1, PER_W - 1)


def kernel(x, weight):
    xf = x.T.reshape(B * H)
    wv = weight.reshape(500000, 128)
    out5 = _emb_lookup(xf, wv)
    return out5.transpose((2, 4, 0, 1, 3)).reshape(B, H, D)
